# baseline (speedup 1.0000x reference)
import functools

import jax
import jax.numpy as jnp
from jax import lax
from jax.experimental import pallas as pl
from jax.experimental.pallas import tpu as pltpu
from jax.experimental.pallas import tpu_sc as plsc

BATCH = 16384
EMBED = 64
SLAB = 2 * EMBED
NC, NS = 2, 16
NW = NC * NS
B_PER_W = BATCH // NW
CHUNK = 256


def _sc_gather_slabs(uidx, iidx, utab2, itab2):
    mesh = plsc.VectorSubcoreMesh(core_axis_name="c", subcore_axis_name="s")
    slab = jax.ShapeDtypeStruct((BATCH, SLAB), jnp.float32)

    @functools.partial(
        pl.kernel,
        mesh=mesh,
        out_type=[slab, slab],
        scratch_types=[
            pltpu.VMEM((B_PER_W,), jnp.int32),
            pltpu.VMEM((B_PER_W,), jnp.int32),
            pltpu.VMEM((CHUNK, SLAB), jnp.float32),
            pltpu.VMEM((CHUNK, SLAB), jnp.float32),
            pltpu.SemaphoreType.DMA,
            pltpu.SemaphoreType.DMA,
        ],
    )
    def gather_k(uidx_hbm, iidx_hbm, utab_hbm, itab_hbm, uout_hbm, iout_hbm,
                 uidx_v, iidx_v, ubuf, ibuf, usem, isem):
        wid = lax.axis_index("s") * NC + lax.axis_index("c")
        base = wid * B_PER_W
        pltpu.sync_copy(uidx_hbm.at[pl.ds(base, B_PER_W)], uidx_v)
        pltpu.sync_copy(iidx_hbm.at[pl.ds(base, B_PER_W)], iidx_v)
        for h in range(B_PER_W // CHUNK):
            cu = pltpu.async_copy(
                utab_hbm.at[uidx_v.at[pl.ds(h * CHUNK, CHUNK)]], ubuf, usem)
            ci = pltpu.async_copy(
                itab_hbm.at[iidx_v.at[pl.ds(h * CHUNK, CHUNK)]], ibuf, isem)
            cu.wait()
            pltpu.sync_copy(ubuf, uout_hbm.at[pl.ds(base + h * CHUNK, CHUNK)])
            ci.wait()
            pltpu.sync_copy(ibuf, iout_hbm.at[pl.ds(base + h * CHUNK, CHUNK)])

    return gather_k(uidx, iidx, utab2, itab2)


def _mlp_body(us_ref, is_ref, up_ref, ip_ref, w1u_ref, w1v_ref, b1_ref,
              w2_ref, b2_ref, w3_ref, b3_ref, w4_ref, b4_ref, o_ref):
    us = us_ref[...]
    vs = is_ref[...]
    u = jnp.where(up_ref[...] == 1, us[:, EMBED:], us[:, :EMBED])
    v = jnp.where(ip_ref[...] == 1, vs[:, EMBED:], vs[:, :EMBED])
    h = jnp.dot(u, w1u_ref[...], preferred_element_type=jnp.float32)
    h += jnp.dot(v, w1v_ref[...], preferred_element_type=jnp.float32)
    h = jnp.maximum(h + b1_ref[...], 0.0)
    h = jnp.dot(h, w2_ref[...], preferred_element_type=jnp.float32)
    h = jnp.maximum(h + b2_ref[...], 0.0)
    h = jnp.dot(h, w3_ref[...], preferred_element_type=jnp.float32)
    h = jnp.maximum(h + b3_ref[...], 0.0)
    z = jnp.dot(h, w4_ref[...], preferred_element_type=jnp.float32) + b4_ref[...]
    o_ref[...] = jax.nn.sigmoid(z)


def _tc_mlp(u_slab, i_slab, u_par, i_par, W1, b1, W2, b2, W3, b3, W4, b4):
    BB = 2048
    grid = (BATCH // BB,)
    w1u = W1[:, :EMBED].T
    w1v = W1[:, EMBED:].T
    w2t = W2.T
    w3t = W3.T
    w4t = W4.T
    full = lambda shape: pl.BlockSpec(shape, lambda i: (0, 0))
    out = pl.pallas_call(
        _mlp_body,
        grid=grid,
        in_specs=[
            pl.BlockSpec((BB, SLAB), lambda i: (i, 0)),
            pl.BlockSpec((BB, SLAB), lambda i: (i, 0)),
            pl.BlockSpec((BB, 1), lambda i: (i, 0)),
            pl.BlockSpec((BB, 1), lambda i: (i, 0)),
            full(w1u.shape),
            full(w1v.shape),
            full((1, 128)),
            full(w2t.shape),
            full((1, 64)),
            full(w3t.shape),
            full((1, 32)),
            full(w4t.shape),
            full((1, 1)),
        ],
        out_specs=pl.BlockSpec((BB, 1), lambda i: (i, 0)),
        out_shape=jax.ShapeDtypeStruct((BATCH, 1), jnp.float32),
    )(u_slab, i_slab, u_par, i_par, w1u, w1v, b1.reshape(1, -1), w2t,
      b2.reshape(1, -1), w3t, b3.reshape(1, -1), w4t, b4.reshape(1, 1))
    return jnp.squeeze(out, axis=-1)


def kernel(user_ids, item_ids, user_table, item_table,
           W1, b1, W2, b2, W3, b3, W4, b4):
    uids = user_ids.astype(jnp.int32)
    iids = item_ids.astype(jnp.int32)
    utab2 = user_table.reshape(-1, SLAB)
    itab2 = item_table.reshape(-1, SLAB)
    u_slab, i_slab = _sc_gather_slabs(uids >> 1, iids >> 1, utab2, itab2)
    u_par = (uids & 1).reshape(BATCH, 1)
    i_par = (iids & 1).reshape(BATCH, 1)
    return _tc_mlp(u_slab, i_slab, u_par, i_par,
                   W1, b1, W2, b2, W3, b3, W4, b4)


# speedup vs baseline: 1.0092x; 1.0092x over previous
import functools

import jax
import jax.numpy as jnp
from jax import lax
from jax.experimental import pallas as pl
from jax.experimental.pallas import tpu as pltpu
from jax.experimental.pallas import tpu_sc as plsc

BATCH = 16384
EMBED = 64
SLAB = 2 * EMBED
NC, NS = 2, 16
NW = NC * NS
B_PER_W = BATCH // NW
CW = 128
NCHUNK = B_PER_W // CW


def _sc_gather(uidx2, iidx2, utab2, itab2):
    mesh = plsc.VectorSubcoreMesh(core_axis_name="c", subcore_axis_name="s")
    out = jax.ShapeDtypeStruct((BATCH, SLAB), jnp.float32)

    @functools.partial(
        pl.kernel,
        mesh=mesh,
        out_type=[out, out],
        scratch_types=[
            pltpu.VMEM((NCHUNK, CW), jnp.int32),
            pltpu.VMEM((NCHUNK, CW), jnp.int32),
            pltpu.VMEM((CW, SLAB), jnp.float32),
            pltpu.VMEM((CW, SLAB), jnp.float32),
            pltpu.SemaphoreType.DMA,
            pltpu.SemaphoreType.DMA,
        ],
    )
    def gather_k(uidx_hbm, iidx_hbm, utab_hbm, itab_hbm, uout_hbm, iout_hbm,
                 uidx_v, iidx_v, ubuf, ibuf, usem, isem):
        wid = lax.axis_index("s") * NC + lax.axis_index("c")
        base = wid * B_PER_W
        pltpu.sync_copy(uidx_hbm.at[pl.ds(wid * NCHUNK, NCHUNK)], uidx_v)
        pltpu.sync_copy(iidx_hbm.at[pl.ds(wid * NCHUNK, NCHUNK)], iidx_v)

        for j in range(NCHUNK):
            ucp = pltpu.async_copy(utab_hbm.at[uidx_v.at[j]], ubuf, usem)
            icp = pltpu.async_copy(itab_hbm.at[iidx_v.at[j]], ibuf, isem)
            ucp.wait()
            pltpu.sync_copy(ubuf, uout_hbm.at[pl.ds(base + j * CW, CW)])
            icp.wait()
            pltpu.sync_copy(ibuf, iout_hbm.at[pl.ds(base + j * CW, CW)])

    return gather_k(uidx2, iidx2, utab2, itab2)


def _mlp_body(us_ref, is_ref, pu_ref, pi_ref, w1u_ref, w1v_ref, b1_ref,
              w2_ref, b2_ref, w3_ref, b3_ref, w4_ref, b4_ref, o_ref):
    us = us_ref[...]
    vs = is_ref[...]
    u = jnp.where(pu_ref[...] != 0, us[:, EMBED:], us[:, :EMBED])
    v = jnp.where(pi_ref[...] != 0, vs[:, EMBED:], vs[:, :EMBED])
    h = jnp.dot(u, w1u_ref[...], preferred_element_type=jnp.float32)
    h += jnp.dot(v, w1v_ref[...], preferred_element_type=jnp.float32)
    h = jnp.maximum(h + b1_ref[...], 0.0)
    h = jnp.dot(h, w2_ref[...], preferred_element_type=jnp.float32)
    h = jnp.maximum(h + b2_ref[...], 0.0)
    h = jnp.dot(h, w3_ref[...], preferred_element_type=jnp.float32)
    h = jnp.maximum(h + b3_ref[...], 0.0)
    z = jnp.dot(h, w4_ref[...], preferred_element_type=jnp.float32) + b4_ref[...]
    o_ref[...] = jax.nn.sigmoid(z)


def _tc_mlp(u_slab, i_slab, pu, pi, W1, b1, W2, b2, W3, b3, W4, b4):
    BB = 2048
    grid = (BATCH // BB,)
    w1u = W1[:, :EMBED].T
    w1v = W1[:, EMBED:].T
    w2t = W2.T
    w3t = W3.T
    w4t = W4.T
    full = lambda shape: pl.BlockSpec(shape, lambda i: (0, 0))
    out = pl.pallas_call(
        _mlp_body,
        grid=grid,
        in_specs=[
            pl.BlockSpec((BB, SLAB), lambda i: (i, 0)),
            pl.BlockSpec((BB, SLAB), lambda i: (i, 0)),
            pl.BlockSpec((BB, 1), lambda i: (i, 0)),
            pl.BlockSpec((BB, 1), lambda i: (i, 0)),
            full(w1u.shape),
            full(w1v.shape),
            full((1, 128)),
            full(w2t.shape),
            full((1, 64)),
            full(w3t.shape),
            full((1, 32)),
            full(w4t.shape),
            full((1, 1)),
        ],
        out_specs=pl.BlockSpec((BB, 1), lambda i: (i, 0)),
        out_shape=jax.ShapeDtypeStruct((BATCH, 1), jnp.float32),
    )(u_slab, i_slab, pu, pi, w1u, w1v, b1.reshape(1, -1), w2t,
      b2.reshape(1, -1), w3t, b3.reshape(1, -1), w4t, b4.reshape(1, 1))
    return jnp.squeeze(out, axis=-1)


def kernel(user_ids, item_ids, user_table, item_table,
           W1, b1, W2, b2, W3, b3, W4, b4):
    uids = user_ids.astype(jnp.int32)
    iids = item_ids.astype(jnp.int32)
    uidx2 = (uids >> 1).reshape(NW * NCHUNK, CW)
    iidx2 = (iids >> 1).reshape(NW * NCHUNK, CW)
    pu = (uids & 1).reshape(BATCH, 1)
    pi = (iids & 1).reshape(BATCH, 1)
    utab2 = user_table.reshape(-1, SLAB)
    itab2 = item_table.reshape(-1, SLAB)
    u_slab, i_slab = _sc_gather(uidx2, iidx2, utab2, itab2)
    return _tc_mlp(u_slab, i_slab, pu, pi, W1, b1, W2, b2, W3, b3, W4, b4)
